# baseline (speedup 1.0000x reference)
import math

import jax
import jax.numpy as jnp
from jax.experimental import pallas as pl
from jax.experimental.pallas import tpu as pltpu

_N = 1024
_H = 64
_CUTS = 8
_NRBF = 8
_PROJ = 4
_RADIUS = 6.0
_DK = 8
_HEADS = _H // _DK
_BI = 128
_CENTERS = [2.0 * k / (_NRBF - 1) for k in range(_NRBF)]


def _leaky(x):
    return jnp.where(x >= 0, x, 0.2 * x)


def _gn_full(x, gamma, beta, groups=4, eps=1e-5):
    cpg = x.shape[1] // groups
    segs = []
    for g in range(groups):
        seg = x[:, g * cpg:(g + 1) * cpg]
        mu = jnp.mean(seg)
        var = jnp.mean((seg - mu) ** 2)
        segs.append((seg - mu) / jnp.sqrt(var + eps))
    xn = jnp.concatenate(segs, axis=1)
    return xn * gamma + beta


def _pre_kernel(f_ref, w1t_ref, b1_ref, w2t_ref, b2_ref, gin_ref, bein_ref,
                wqt_ref, bq_ref, wk_ref, bkc_ref, wv_ref, bvc_ref,
                pts_ref, nuv9_ref,
                q_out, kt_out, vt_out, ptss_out, ptsT_out, normT_out):
    ps = pts_ref[...] * (1.0 / (math.sqrt(2.0) * _RADIUS))
    ptss_out[...] = ps
    ptsT_out[...] = ps.T
    normT_out[...] = nuv9_ref[:, :3].T
    f = f_ref[...]
    h = _leaky(jnp.dot(f, w1t_ref[...], preferred_element_type=jnp.float32)
               + b1_ref[...])
    h = _leaky(jnp.dot(h, w2t_ref[...], preferred_element_type=jnp.float32)
               + b2_ref[...])
    h = _gn_full(h, gin_ref[...], bein_ref[...])
    q_out[...] = (jnp.dot(h, wqt_ref[...], preferred_element_type=jnp.float32)
                  + bq_ref[...]) * (1.0 / math.sqrt(_DK))
    ht = h.T
    kt_out[...] = jnp.dot(wk_ref[...], ht, preferred_element_type=jnp.float32) + bkc_ref[...]
    vt_out[...] = (jnp.dot(wv_ref[...], ht, preferred_element_type=jnp.float32)
                   + bvc_ref[...]).astype(jnp.bfloat16)


def _pair_kernel(pts_i_ref, ptsT_ref, nuv9_ref, normT_ref, q_ref, kT_ref, vT_ref,
                 A1_ref, B1_ref, A2_ref, B2_ref, Wr_ref, br_ref, sc_ref,
                 out_ref):
    di = [ptsT_ref[b:b + 1, :] - pts_i_ref[:, b:b + 1] for b in range(3)]
    nn = (nuv9_ref[:, 0:1] * normT_ref[0:1, :]
          + nuv9_ref[:, 1:2] * normT_ref[1:2, :]
          + nuv9_ref[:, 2:3] * normT_ref[2:3, :])
    d2 = (di[0] * di[0] + di[1] * di[1] + di[2] * di[2]) * (2.0 - nn) ** 2
    t = 1.0 + d2 * (1.0 / 3.0)
    wij = 1.0 / (t * t * t)
    bd = jnp.sqrt(jnp.maximum(d2, 1e-12))

    neg_inv2s2 = sc_ref[0, 0]
    m_coef = sc_ref[0, 1]
    cap = sc_ref[0, 2]
    bdc = jnp.minimum(bd, cap)
    Rk = [jnp.exp(bdc * bdc * neg_inv2s2)]
    M = jnp.exp(bdc * m_coef)
    for k in range(1, _NRBF):
        Rk.append(Rk[k - 1] * M * sc_ref[0, 2 + k])

    bf16 = jnp.bfloat16
    di16 = [x.astype(bf16) for x in di]
    R16 = [r.astype(bf16) for r in Rk]
    X = []
    for a in range(3):
        X.append(nuv9_ref[:, 3 * a:3 * a + 1].astype(bf16) * di16[0]
                 + nuv9_ref[:, 3 * a + 1:3 * a + 2].astype(bf16) * di16[1]
                 + nuv9_ref[:, 3 * a + 2:3 * a + 3].astype(bf16) * di16[2])
    X.append(nn.astype(bf16))
    for p in range(_PROJ):
        t0 = R16[0] * Wr_ref[p, 0].astype(bf16) + R16[1] * Wr_ref[p, 1].astype(bf16)
        t1 = R16[2] * Wr_ref[p, 2].astype(bf16) + R16[3] * Wr_ref[p, 3].astype(bf16)
        t2 = R16[4] * Wr_ref[p, 4].astype(bf16) + R16[5] * Wr_ref[p, 5].astype(bf16)
        t3 = R16[6] * Wr_ref[p, 6].astype(bf16) + R16[7] * Wr_ref[p, 7].astype(bf16)
        X.append((t0 + t1) + (t2 + t3 + br_ref[0, p].astype(bf16)))

    Fp16 = []
    for c in range(_CUTS):
        a = [A1_ref[c, x].astype(bf16) for x in range(8)]
        t0 = X[0] * a[0] + X[1] * a[1]
        t1 = X[2] * a[2] + X[3] * a[3]
        t2 = X[4] * a[4] + X[5] * a[5]
        t3 = X[6] * a[6] + X[7] * a[7]
        s = (t0 + t1) + (t2 + t3 + B1_ref[0, c].astype(bf16))
        Fp16.append(jnp.maximum(s, bf16(0.0)))

    P = []
    rz = []
    for h in range(_HEADS):
        qh = q_ref[:, h * _DK:(h + 1) * _DK]
        kh = kT_ref[h * _DK:(h + 1) * _DK, :]
        S = jnp.dot(qh, kh, preferred_element_type=jnp.float32)
        e = jnp.exp(S)
        rz.append(1.0 / jnp.sum(e, axis=1, keepdims=True))
        P.append((e * wij).astype(bf16))

    cols = []
    half = _N // 2
    for d in range(_H):
        h = d // _DK
        a = [A2_ref[d, c].astype(bf16) for c in range(_CUTS)]
        b2 = B2_ref[0, d].astype(bf16)
        parts = []
        qtr = _N // 4
        for j0 in (0, qtr, 2 * qtr, 3 * qtr):
            Fc = [f[:, j0:j0 + qtr] for f in Fp16]
            t0 = Fc[0] * a[0] + Fc[1] * a[1]
            t1 = Fc[2] * a[2] + Fc[3] * a[3]
            t2 = Fc[4] * a[4] + Fc[5] * a[5]
            t3 = Fc[6] * a[6] + Fc[7] * a[7]
            s = (t0 + t1) + (t2 + t3 + b2)
            Fd = jnp.maximum(s, bf16(0.0))
            parts.append((P[h][:, j0:j0 + qtr] * Fd)
                         * vT_ref[d:d + 1, j0:j0 + qtr])
        part = (parts[0] + parts[1]) + (parts[2] + parts[3])
        cols.append(jnp.sum(part.astype(jnp.float32), axis=1,
                            keepdims=True) * rz[h])
    out_ref[...] = jnp.concatenate(cols, axis=1)


def _post_kernel(agg_ref, f_ref, wo1t_ref, bo1_ref, wo2t_ref, bo2_ref,
                 gout_ref, beout_ref, wrest_ref, bres_ref, out_ref):
    a = agg_ref[...]
    o = _leaky(jnp.dot(a, wo1t_ref[...], preferred_element_type=jnp.float32)
               + bo1_ref[...])
    o = _leaky(jnp.dot(o, wo2t_ref[...], preferred_element_type=jnp.float32)
               + bo2_ref[...])
    o = _gn_full(o, gout_ref[...], beout_ref[...])
    out_ref[...] = (o + jnp.dot(f_ref[...], wrest_ref[...],
                                preferred_element_type=jnp.float32)
                    + bres_ref[...])


def kernel(points, nuv, features, W1, b1, W2, b2, g_in, be_in, Wq, bq, Wk, bk,
           Wv, bv, rls, Wr_, br_, A1, B1, A2, B2, Wo1, bo1, Wo2, bo2, g_out,
           be_out, Wres, bres):
    f32 = jnp.float32
    row = lambda v: v.reshape(1, -1).astype(f32)

    col = lambda vv: vv.reshape(-1, 1).astype(f32)
    nuv9 = nuv.reshape(_N, 9)
    q, kT, vT, pts, ptsT, normT = pl.pallas_call(
        _pre_kernel,
        out_shape=[jax.ShapeDtypeStruct((_N, _H), f32),
                   jax.ShapeDtypeStruct((_H, _N), f32),
                   jax.ShapeDtypeStruct((_H, _N), jnp.bfloat16),
                   jax.ShapeDtypeStruct((_N, 3), f32),
                   jax.ShapeDtypeStruct((3, _N), f32),
                   jax.ShapeDtypeStruct((3, _N), f32)],
    )(features, W1.T, row(b1), W2.T, row(b2), row(g_in), row(be_in),
      Wq.T, row(bq), Wk, col(bk), Wv, col(bv), points, nuv9)

    sigma = jnp.maximum(jnp.exp(rls), 1e-6)
    inv_s2 = 1.0 / (sigma * sigma)
    delta = 2.0 / (_NRBF - 1)
    sc_consts = jnp.concatenate([
        (-0.5 * inv_s2).reshape(1),
        (delta * inv_s2).reshape(1),
        (2.0 + 12.0 * sigma).reshape(1),
        jnp.stack([jnp.exp(-((k - 1) * delta * delta + 0.5 * delta * delta)
                           * inv_s2) for k in range(1, _NRBF)]),
    ]).reshape(1, 2 + _NRBF).astype(f32)

    grid = (_N // _BI,)
    smem = pl.BlockSpec(memory_space=pltpu.SMEM)
    agg = pl.pallas_call(
        _pair_kernel,
        grid=grid,
        in_specs=[
            pl.BlockSpec((_BI, 3), lambda i: (i, 0)),
            pl.BlockSpec((3, _N), lambda i: (0, 0)),
            pl.BlockSpec((_BI, 9), lambda i: (i, 0)),
            pl.BlockSpec((3, _N), lambda i: (0, 0)),
            pl.BlockSpec((_BI, _H), lambda i: (i, 0)),
            pl.BlockSpec((_H, _N), lambda i: (0, 0)),
            pl.BlockSpec((_H, _N), lambda i: (0, 0)),
            smem, smem, smem, smem, smem, smem, smem,
        ],
        out_specs=pl.BlockSpec((_BI, _H), lambda i: (i, 0)),
        out_shape=jax.ShapeDtypeStruct((_N, _H), f32),
        compiler_params=pltpu.CompilerParams(
            dimension_semantics=("parallel",)),
    )(pts, ptsT, nuv9, normT, q, kT, vT,
      A1, row(B1), A2, row(B2), Wr_, row(br_), sc_consts)

    out = pl.pallas_call(
        _post_kernel,
        out_shape=jax.ShapeDtypeStruct((_N, _H), f32),
    )(agg, features, Wo1.T, row(bo1), Wo2.T, row(bo2), row(g_out),
      row(be_out), Wres.T, row(bres))
    return out


# speedup vs baseline: 1.0006x; 1.0006x over previous
import math

import jax
import jax.numpy as jnp
from jax.experimental import pallas as pl
from jax.experimental.pallas import tpu as pltpu

_N = 1024
_H = 64
_CUTS = 8
_NRBF = 8
_PROJ = 4
_RADIUS = 6.0
_DK = 8
_HEADS = _H // _DK
_BI = 128
_CENTERS = [2.0 * k / (_NRBF - 1) for k in range(_NRBF)]


def _leaky(x):
    return jnp.where(x >= 0, x, 0.2 * x)


def _gn_full(x, gamma, beta, groups=4, eps=1e-5):
    cpg = x.shape[1] // groups
    segs = []
    for g in range(groups):
        seg = x[:, g * cpg:(g + 1) * cpg]
        mu = jnp.mean(seg)
        var = jnp.mean((seg - mu) ** 2)
        segs.append((seg - mu) / jnp.sqrt(var + eps))
    xn = jnp.concatenate(segs, axis=1)
    return xn * gamma + beta


def _pre_kernel(f_ref, w1t_ref, b1_ref, w2t_ref, b2_ref, gin_ref, bein_ref,
                wqt_ref, bq_ref, wk_ref, bkc_ref, wv_ref, bvc_ref,
                pts_ref, nuv9_ref,
                q_out, kt_out, vt_out, ptss_out, ptsT_out, normT_out):
    ps = pts_ref[...] * (1.0 / (math.sqrt(2.0) * _RADIUS))
    ptss_out[...] = ps
    ptsT_out[...] = ps.T
    normT_out[...] = nuv9_ref[:, :3].T
    f = f_ref[...]
    h = _leaky(jnp.dot(f, w1t_ref[...], preferred_element_type=jnp.float32)
               + b1_ref[...])
    h = _leaky(jnp.dot(h, w2t_ref[...], preferred_element_type=jnp.float32)
               + b2_ref[...])
    h = _gn_full(h, gin_ref[...], bein_ref[...])
    q_out[...] = (jnp.dot(h, wqt_ref[...], preferred_element_type=jnp.float32)
                  + bq_ref[...]) * (1.0 / math.sqrt(_DK))
    ht = h.T
    kt_out[...] = jnp.dot(wk_ref[...], ht, preferred_element_type=jnp.float32) + bkc_ref[...]
    vt_out[...] = (jnp.dot(wv_ref[...], ht, preferred_element_type=jnp.float32)
                   + bvc_ref[...]).astype(jnp.bfloat16)


def _pair_kernel(pts_i_ref, ptsT_ref, nuv9_ref, normT_ref, q_ref, kT_ref, vT_ref,
                 A1_ref, B1_ref, A2_ref, B2_ref, Wr_ref, br_ref, sc_ref,
                 out_ref):
    di = [ptsT_ref[b:b + 1, :] - pts_i_ref[:, b:b + 1] for b in range(3)]
    nn = (nuv9_ref[:, 0:1] * normT_ref[0:1, :]
          + nuv9_ref[:, 1:2] * normT_ref[1:2, :]
          + nuv9_ref[:, 2:3] * normT_ref[2:3, :])
    d2 = (di[0] * di[0] + di[1] * di[1] + di[2] * di[2]) * (2.0 - nn) ** 2
    t = 1.0 + d2 * (1.0 / 3.0)
    wij = 1.0 / (t * t * t)
    bd = jnp.sqrt(jnp.maximum(d2, 1e-12))

    neg_inv2s2 = sc_ref[0, 0]
    m_coef = sc_ref[0, 1]
    cap = sc_ref[0, 2]
    bdc = jnp.minimum(bd, cap)
    Rk = [jnp.exp(bdc * bdc * neg_inv2s2)]
    M = jnp.exp(bdc * m_coef)
    for k in range(1, _NRBF):
        Rk.append(Rk[k - 1] * M * sc_ref[0, 2 + k])

    bf16 = jnp.bfloat16
    di16 = [x.astype(bf16) for x in di]
    R16 = [r.astype(bf16) for r in Rk]
    X = []
    for a in range(3):
        X.append(nuv9_ref[:, 3 * a:3 * a + 1].astype(bf16) * di16[0]
                 + nuv9_ref[:, 3 * a + 1:3 * a + 2].astype(bf16) * di16[1]
                 + nuv9_ref[:, 3 * a + 2:3 * a + 3].astype(bf16) * di16[2])
    X.append(nn.astype(bf16))
    for p in range(_PROJ):
        t0 = R16[0] * Wr_ref[p, 0].astype(bf16) + R16[1] * Wr_ref[p, 1].astype(bf16)
        t1 = R16[2] * Wr_ref[p, 2].astype(bf16) + R16[3] * Wr_ref[p, 3].astype(bf16)
        t2 = R16[4] * Wr_ref[p, 4].astype(bf16) + R16[5] * Wr_ref[p, 5].astype(bf16)
        t3 = R16[6] * Wr_ref[p, 6].astype(bf16) + R16[7] * Wr_ref[p, 7].astype(bf16)
        X.append((t0 + t1) + (t2 + t3 + br_ref[0, p].astype(bf16)))

    Fp16 = []
    for c in range(_CUTS):
        a = [A1_ref[c, x].astype(bf16) for x in range(8)]
        t0 = X[0] * a[0] + X[1] * a[1]
        t1 = X[2] * a[2] + X[3] * a[3]
        t2 = X[4] * a[4] + X[5] * a[5]
        t3 = X[6] * a[6] + X[7] * a[7]
        s = (t0 + t1) + (t2 + t3 + B1_ref[0, c].astype(bf16))
        Fp16.append(jnp.maximum(s, bf16(0.0)))

    P = []
    rz = []
    for h in range(_HEADS):
        qh = q_ref[:, h * _DK:(h + 1) * _DK]
        kh = kT_ref[h * _DK:(h + 1) * _DK, :]
        S = jnp.dot(qh, kh, preferred_element_type=jnp.float32)
        e = jnp.exp(S)
        rz.append(1.0 / jnp.sum(e, axis=1, keepdims=True))
        P.append((e * wij).astype(bf16))

    cols = []
    half = _N // 2
    for d in range(_H):
        h = d // _DK
        a = [A2_ref[d, c].astype(bf16) for c in range(_CUTS)]
        b2 = B2_ref[0, d].astype(bf16)
        parts = []
        for j0 in (0, half):
            Fc = [f[:, j0:j0 + half] for f in Fp16]
            t0 = Fc[0] * a[0] + Fc[1] * a[1]
            t1 = Fc[2] * a[2] + Fc[3] * a[3]
            t2 = Fc[4] * a[4] + Fc[5] * a[5]
            t3 = Fc[6] * a[6] + Fc[7] * a[7]
            s = (t0 + t1) + (t2 + t3 + b2)
            Fd = jnp.maximum(s, bf16(0.0))
            parts.append((P[h][:, j0:j0 + half] * Fd)
                         * vT_ref[d:d + 1, j0:j0 + half])
        contrib = parts[0] + parts[1]
        part = contrib[:, :256] + contrib[:, 256:]
        cols.append(jnp.sum(part.astype(jnp.float32), axis=1,
                            keepdims=True) * rz[h])
    out_ref[...] = jnp.concatenate(cols, axis=1)


def _post_kernel(agg_ref, f_ref, wo1t_ref, bo1_ref, wo2t_ref, bo2_ref,
                 gout_ref, beout_ref, wrest_ref, bres_ref, out_ref):
    a = agg_ref[...]
    o = _leaky(jnp.dot(a, wo1t_ref[...], preferred_element_type=jnp.float32)
               + bo1_ref[...])
    o = _leaky(jnp.dot(o, wo2t_ref[...], preferred_element_type=jnp.float32)
               + bo2_ref[...])
    o = _gn_full(o, gout_ref[...], beout_ref[...])
    out_ref[...] = (o + jnp.dot(f_ref[...], wrest_ref[...],
                                preferred_element_type=jnp.float32)
                    + bres_ref[...])


def kernel(points, nuv, features, W1, b1, W2, b2, g_in, be_in, Wq, bq, Wk, bk,
           Wv, bv, rls, Wr_, br_, A1, B1, A2, B2, Wo1, bo1, Wo2, bo2, g_out,
           be_out, Wres, bres):
    f32 = jnp.float32
    row = lambda v: v.reshape(1, -1).astype(f32)

    col = lambda vv: vv.reshape(-1, 1).astype(f32)
    nuv9 = nuv.reshape(_N, 9)
    q, kT, vT, pts, ptsT, normT = pl.pallas_call(
        _pre_kernel,
        out_shape=[jax.ShapeDtypeStruct((_N, _H), f32),
                   jax.ShapeDtypeStruct((_H, _N), f32),
                   jax.ShapeDtypeStruct((_H, _N), jnp.bfloat16),
                   jax.ShapeDtypeStruct((_N, 3), f32),
                   jax.ShapeDtypeStruct((3, _N), f32),
                   jax.ShapeDtypeStruct((3, _N), f32)],
    )(features, W1.T, row(b1), W2.T, row(b2), row(g_in), row(be_in),
      Wq.T, row(bq), Wk, col(bk), Wv, col(bv), points, nuv9)

    sigma = jnp.maximum(jnp.exp(rls), 1e-6)
    inv_s2 = 1.0 / (sigma * sigma)
    delta = 2.0 / (_NRBF - 1)
    sc_consts = jnp.concatenate([
        (-0.5 * inv_s2).reshape(1),
        (delta * inv_s2).reshape(1),
        (2.0 + 12.0 * sigma).reshape(1),
        jnp.stack([jnp.exp(-((k - 1) * delta * delta + 0.5 * delta * delta)
                           * inv_s2) for k in range(1, _NRBF)]),
    ]).reshape(1, 2 + _NRBF).astype(f32)

    grid = (_N // _BI,)
    smem = pl.BlockSpec(memory_space=pltpu.SMEM)
    agg = pl.pallas_call(
        _pair_kernel,
        grid=grid,
        in_specs=[
            pl.BlockSpec((_BI, 3), lambda i: (i, 0)),
            pl.BlockSpec((3, _N), lambda i: (0, 0)),
            pl.BlockSpec((_BI, 9), lambda i: (i, 0)),
            pl.BlockSpec((3, _N), lambda i: (0, 0)),
            pl.BlockSpec((_BI, _H), lambda i: (i, 0)),
            pl.BlockSpec((_H, _N), lambda i: (0, 0)),
            pl.BlockSpec((_H, _N), lambda i: (0, 0)),
            smem, smem, smem, smem, smem, smem, smem,
        ],
        out_specs=pl.BlockSpec((_BI, _H), lambda i: (i, 0)),
        out_shape=jax.ShapeDtypeStruct((_N, _H), f32),
        compiler_params=pltpu.CompilerParams(
            dimension_semantics=("parallel",)),
    )(pts, ptsT, nuv9, normT, q, kT, vT,
      A1, row(B1), A2, row(B2), Wr_, row(br_), sc_consts)

    out = pl.pallas_call(
        _post_kernel,
        out_shape=jax.ShapeDtypeStruct((_N, _H), f32),
    )(agg, features, Wo1.T, row(bo1), Wo2.T, row(bo2), row(g_out),
      row(be_out), Wres.T, row(bres))
    return out


# speedup vs baseline: 1.0092x; 1.0086x over previous
import math

import jax
import jax.numpy as jnp
from jax.experimental import pallas as pl
from jax.experimental.pallas import tpu as pltpu

_N = 1024
_H = 64
_CUTS = 8
_NRBF = 8
_PROJ = 4
_RADIUS = 6.0
_DK = 8
_HEADS = _H // _DK
_BI = 128


def _leaky(x):
    return jnp.where(x >= 0, x, 0.2 * x)


def _gn_full(x, gamma, beta, groups=4, eps=1e-5):
    cpg = x.shape[1] // groups
    segs = []
    for g in range(groups):
        seg = x[:, g * cpg:(g + 1) * cpg]
        mu = jnp.mean(seg)
        var = jnp.mean((seg - mu) ** 2)
        segs.append((seg - mu) / jnp.sqrt(var + eps))
    xn = jnp.concatenate(segs, axis=1)
    return xn * gamma + beta


def _pre_kernel(f_ref, w1t_ref, b1_ref, w2t_ref, b2_ref, gin_ref, bein_ref,
                wqt_ref, bq_ref, wk_ref, bkc_ref, wv_ref, bvc_ref,
                pts_ref, nuv9_ref,
                q_out, kt_out, vt_out, ptss_out, ptsT_out, normT_out):
    ps = pts_ref[...] * (1.0 / (math.sqrt(2.0) * _RADIUS))
    ptss_out[...] = ps
    ptsT_out[...] = ps.T
    normT_out[...] = nuv9_ref[:, :3].T
    f = f_ref[...]
    h = _leaky(jnp.dot(f, w1t_ref[...], preferred_element_type=jnp.float32)
               + b1_ref[...])
    h = _leaky(jnp.dot(h, w2t_ref[...], preferred_element_type=jnp.float32)
               + b2_ref[...])
    h = _gn_full(h, gin_ref[...], bein_ref[...])
    q_out[...] = (jnp.dot(h, wqt_ref[...], preferred_element_type=jnp.float32)
                  + bq_ref[...]) * (1.0 / math.sqrt(_DK))
    ht = h.T
    kt_out[...] = jnp.dot(wk_ref[...], ht, preferred_element_type=jnp.float32) + bkc_ref[...]
    vt_out[...] = (jnp.dot(wv_ref[...], ht, preferred_element_type=jnp.float32)
                   + bvc_ref[...]).astype(jnp.bfloat16)


def _pair_kernel(pts_i_ref, ptsT_ref, nuv9_ref, normT_ref, q_ref, kT_ref, vT_ref,
                 A1_ref, B1_ref, A2_ref, B2_ref, Wr_ref, br_ref, sc_ref,
                 out_ref):
    di = [ptsT_ref[b:b + 1, :] - pts_i_ref[:, b:b + 1] for b in range(3)]
    nn = (nuv9_ref[:, 0:1] * normT_ref[0:1, :]
          + nuv9_ref[:, 1:2] * normT_ref[1:2, :]
          + nuv9_ref[:, 2:3] * normT_ref[2:3, :])
    d2 = (di[0] * di[0] + di[1] * di[1] + di[2] * di[2]) * (2.0 - nn) ** 2
    t = 1.0 + d2 * (1.0 / 3.0)
    wij = 1.0 / (t * t * t)
    bd = jnp.sqrt(jnp.maximum(d2, 1e-12))

    neg_inv2s2 = sc_ref[0, 0]
    m_coef = sc_ref[0, 1]
    cap = sc_ref[0, 2]
    bdc = jnp.minimum(bd, cap)
    Rk = [jnp.exp(bdc * bdc * neg_inv2s2)]
    M = jnp.exp(bdc * m_coef)
    for k in range(1, _NRBF):
        Rk.append(Rk[k - 1] * M * sc_ref[0, 2 + k])

    bf16 = jnp.bfloat16
    di16 = [x.astype(bf16) for x in di]
    R16 = [r.astype(bf16) for r in Rk]
    X = []
    for a in range(3):
        X.append(nuv9_ref[:, 3 * a:3 * a + 1].astype(bf16) * di16[0]
                 + nuv9_ref[:, 3 * a + 1:3 * a + 2].astype(bf16) * di16[1]
                 + nuv9_ref[:, 3 * a + 2:3 * a + 3].astype(bf16) * di16[2])
    X.append(nn.astype(bf16))
    for p in range(_PROJ):
        t0 = R16[0] * Wr_ref[p, 0].astype(bf16) + R16[1] * Wr_ref[p, 1].astype(bf16)
        t1 = R16[2] * Wr_ref[p, 2].astype(bf16) + R16[3] * Wr_ref[p, 3].astype(bf16)
        t2 = R16[4] * Wr_ref[p, 4].astype(bf16) + R16[5] * Wr_ref[p, 5].astype(bf16)
        t3 = R16[6] * Wr_ref[p, 6].astype(bf16) + R16[7] * Wr_ref[p, 7].astype(bf16)
        X.append((t0 + t1) + (t2 + t3 + br_ref[0, p].astype(bf16)))

    Fp16 = []
    for c in range(_CUTS):
        a = [A1_ref[c, x].astype(bf16) for x in range(8)]
        t0 = X[0] * a[0] + X[1] * a[1]
        t1 = X[2] * a[2] + X[3] * a[3]
        t2 = X[4] * a[4] + X[5] * a[5]
        t3 = X[6] * a[6] + X[7] * a[7]
        s = (t0 + t1) + (t2 + t3 + B1_ref[0, c].astype(bf16))
        Fp16.append(jnp.maximum(s, bf16(0.0)))

    P = []
    rz = []
    for h in range(_HEADS):
        qh = q_ref[:, h * _DK:(h + 1) * _DK]
        kh = kT_ref[h * _DK:(h + 1) * _DK, :]
        S = jnp.dot(qh, kh, preferred_element_type=jnp.float32)
        e = jnp.exp(S)
        rz.append(1.0 / jnp.sum(e, axis=1, keepdims=True))
        P.append((e * wij).astype(bf16))

    cols = []
    half = _N // 2
    for d in range(_H):
        h = d // _DK
        a = [A2_ref[d, c].astype(bf16) for c in range(_CUTS)]
        b2 = B2_ref[0, d].astype(bf16)
        parts = []
        for j0 in (0, half):
            Fc = [f[:, j0:j0 + half] for f in Fp16]
            t0 = Fc[0] * a[0] + Fc[1] * a[1]
            t1 = Fc[2] * a[2] + Fc[3] * a[3]
            t2 = Fc[4] * a[4] + Fc[5] * a[5]
            t3 = Fc[6] * a[6] + Fc[7] * a[7]
            s = (t0 + t1) + (t2 + t3 + b2)
            Fd = jnp.maximum(s, bf16(0.0))
            parts.append((P[h][:, j0:j0 + half] * Fd)
                         * vT_ref[d:d + 1, j0:j0 + half])
        contrib = parts[0] + parts[1]
        part = contrib[:, :256] + contrib[:, 256:]
        out_ref[:, d:d + 1] = jnp.sum(part.astype(jnp.float32), axis=1,
                                      keepdims=True) * rz[h]


def _post_kernel(agg_ref, f_ref, wo1t_ref, bo1_ref, wo2t_ref, bo2_ref,
                 gout_ref, beout_ref, wrest_ref, bres_ref, out_ref):
    a = agg_ref[...]
    o = _leaky(jnp.dot(a, wo1t_ref[...], preferred_element_type=jnp.float32)
               + bo1_ref[...])
    o = _leaky(jnp.dot(o, wo2t_ref[...], preferred_element_type=jnp.float32)
               + bo2_ref[...])
    o = _gn_full(o, gout_ref[...], beout_ref[...])
    out_ref[...] = (o + jnp.dot(f_ref[...], wrest_ref[...],
                                preferred_element_type=jnp.float32)
                    + bres_ref[...])


def kernel(points, nuv, features, W1, b1, W2, b2, g_in, be_in, Wq, bq, Wk, bk,
           Wv, bv, rls, Wr_, br_, A1, B1, A2, B2, Wo1, bo1, Wo2, bo2, g_out,
           be_out, Wres, bres):
    f32 = jnp.float32
    row = lambda v: v.reshape(1, -1).astype(f32)

    col = lambda vv: vv.reshape(-1, 1).astype(f32)
    nuv9 = nuv.reshape(_N, 9)
    q, kT, vT, pts, ptsT, normT = pl.pallas_call(
        _pre_kernel,
        out_shape=[jax.ShapeDtypeStruct((_N, _H), f32),
                   jax.ShapeDtypeStruct((_H, _N), f32),
                   jax.ShapeDtypeStruct((_H, _N), jnp.bfloat16),
                   jax.ShapeDtypeStruct((_N, 3), f32),
                   jax.ShapeDtypeStruct((3, _N), f32),
                   jax.ShapeDtypeStruct((3, _N), f32)],
    )(features, W1.T, row(b1), W2.T, row(b2), row(g_in), row(be_in),
      Wq.T, row(bq), Wk, col(bk), Wv, col(bv), points, nuv9)

    sigma = jnp.maximum(jnp.exp(rls), 1e-6)
    inv_s2 = 1.0 / (sigma * sigma)
    delta = 2.0 / (_NRBF - 1)
    sc_consts = jnp.concatenate([
        (-0.5 * inv_s2).reshape(1),
        (delta * inv_s2).reshape(1),
        (2.0 + 12.0 * sigma).reshape(1),
        jnp.stack([jnp.exp(-((k - 1) * delta * delta + 0.5 * delta * delta)
                           * inv_s2) for k in range(1, _NRBF)]),
    ]).reshape(1, 2 + _NRBF).astype(f32)

    grid = (_N // _BI,)
    smem = pl.BlockSpec(memory_space=pltpu.SMEM)
    agg = pl.pallas_call(
        _pair_kernel,
        grid=grid,
        in_specs=[
            pl.BlockSpec((_BI, 3), lambda i: (i, 0)),
            pl.BlockSpec((3, _N), lambda i: (0, 0)),
            pl.BlockSpec((_BI, 9), lambda i: (i, 0)),
            pl.BlockSpec((3, _N), lambda i: (0, 0)),
            pl.BlockSpec((_BI, _H), lambda i: (i, 0)),
            pl.BlockSpec((_H, _N), lambda i: (0, 0)),
            pl.BlockSpec((_H, _N), lambda i: (0, 0)),
            smem, smem, smem, smem, smem, smem, smem,
        ],
        out_specs=pl.BlockSpec((_BI, _H), lambda i: (i, 0)),
        out_shape=jax.ShapeDtypeStruct((_N, _H), f32),
        compiler_params=pltpu.CompilerParams(
            dimension_semantics=("parallel",)),
    )(pts, ptsT, nuv9, normT, q, kT, vT,
      A1, row(B1), A2, row(B2), Wr_, row(br_), sc_consts)

    out = pl.pallas_call(
        _post_kernel,
        out_shape=jax.ShapeDtypeStruct((_N, _H), f32),
    )(agg, features, Wo1.T, row(bo1), Wo2.T, row(bo2), row(g_out),
      row(be_out), Wres.T, row(bres))
    return out
